# baseline (speedup 1.0000x reference)
import functools

import jax
import jax.numpy as jnp
from jax import lax
from jax.experimental import pallas as pl
from jax.experimental.pallas import tpu as pltpu
from jax.experimental.pallas import tpu_sc as plsc

N = 10000
E = 320000
D = 128
H = 512

NC = 2
NS = 16
NW = NC * NS

SCALE = 256.0
CH = 128
EPT = E // NW
NCH = 80
EPT_P = NCH * CH
TRASH = N
ACC_ROWS = 10240
ZR = ACC_ROWS // NS


def _sc_segment_sum(x_q, src_r, dst_r, zrows):
  mesh = plsc.VectorSubcoreMesh(
      core_axis_name="c", subcore_axis_name="s", num_cores=NC,
      num_subcores=NS)

  @functools.partial(
      pl.kernel,
      out_type=jax.ShapeDtypeStruct((NC, ACC_ROWS, D), jnp.int16),
      mesh=mesh,
      scratch_types=[
          pltpu.VMEM((NCH, CH), jnp.int32),
          pltpu.VMEM((NCH, CH), jnp.int32),
          pltpu.VMEM((CH, D), jnp.int16),
          pltpu.VMEM((CH, D), jnp.int16),
          pltpu.VMEM((CH, D), jnp.int16),
          pltpu.VMEM((CH, D), jnp.int16),
          pltpu.SemaphoreType.DMA,
          pltpu.SemaphoreType.DMA,
          pltpu.SemaphoreType.DMA,
          pltpu.SemaphoreType.DMA,
          pltpu.SemaphoreType.DMA,
          pltpu.SemaphoreType.DMA,
          pltpu.SemaphoreType.DMA,
          pltpu.SemaphoreType.DMA,
          pltpu.VMEM_SHARED((ACC_ROWS, D), jnp.int16),
      ],
      compiler_params=pltpu.CompilerParams(use_tc_tiling_on_sc=False),
  )
  def seg_sum(x_hbm, src_hbm, dst_hbm, zero_hbm, out_hbm,
              src_v, dst_v, g0, g1, g2, g3,
              gs0, gs1, gs2, gs3, ss0, ss1, ss2, ss3, acc):
    c = lax.axis_index("c")
    s = lax.axis_index("s")
    wid = s * NC + c
    gb = [g0, g1, g2, g3]
    gsem = [gs0, gs1, gs2, gs3]
    ssem = [ss0, ss1, ss2, ss3]

    pltpu.sync_copy(src_hbm.at[wid], src_v)
    pltpu.sync_copy(dst_hbm.at[wid], dst_v)
    pltpu.sync_copy(zero_hbm, acc.at[pl.ds(s * ZR, ZR)])
    plsc.subcore_barrier()

    for t in range(3):
      pltpu.async_copy(x_hbm.at[src_v.at[t]], gb[t], gsem[t])

    def quad(p, carry):
      for t in range(4):
        j = 4 * p + t

        @pl.when(j >= 4)
        def _drain_scatter():
          pltpu.make_async_copy(gb[t], acc.at[dst_v.at[j]], ssem[t]).wait()

        pltpu.make_async_copy(x_hbm.at[src_v.at[j]], gb[t], gsem[t]).wait()
        pltpu.async_copy(gb[t], acc.at[dst_v.at[j]], ssem[t], add=True)

        @pl.when(j + 3 < NCH)
        def _fire_next():
          tn = (t + 3) % 4
          pltpu.async_copy(x_hbm.at[src_v.at[j + 3]], gb[tn], gsem[tn])

      return carry

    lax.fori_loop(0, NCH // 4, quad, None)
    for t in range(4):
      pltpu.make_async_copy(gb[t], acc.at[dst_v.at[NCH - 4 + t]],
                            ssem[t]).wait()
    plsc.subcore_barrier()

    pltpu.sync_copy(acc.at[pl.ds(s * ZR, ZR)],
                    out_hbm.at[c, pl.ds(s * ZR, ZR)])

  return seg_sum(x_q, src_r, dst_r, zrows)


BM = 1000


def _mlp_body(u_ref, x_ref, w1_ref, b1_ref, w2_ref, b2_ref, g_ref, bt_ref,
              o_ref):
  u = (u_ref[0].astype(jnp.float32) + u_ref[1].astype(jnp.float32))
  u = u * (1.0 / SCALE)
  h1 = jnp.dot(u, w1_ref[...], preferred_element_type=jnp.float32)
  h1 = jnp.maximum(h1 + b1_ref[...], 0.0)
  h = jnp.dot(h1, w2_ref[...], preferred_element_type=jnp.float32)
  y = h + b2_ref[...] + x_ref[...]
  mean = jnp.mean(y, axis=1, keepdims=True)
  yc = y - mean
  var = jnp.mean(yc * yc, axis=1, keepdims=True)
  o_ref[...] = yc * lax.rsqrt(var + 1e-5) * g_ref[...] + bt_ref[...]


def _mlp_ln(u_part, x, W1, b1, W2, b2, gamma, beta):
  grid = (N // BM,)
  return pl.pallas_call(
      _mlp_body,
      grid=grid,
      in_specs=[
          pl.BlockSpec((NC, BM, D), lambda i: (0, i, 0)),
          pl.BlockSpec((BM, D), lambda i: (i, 0)),
          pl.BlockSpec((D, H), lambda i: (0, 0)),
          pl.BlockSpec((1, H), lambda i: (0, 0)),
          pl.BlockSpec((H, D), lambda i: (0, 0)),
          pl.BlockSpec((1, D), lambda i: (0, 0)),
          pl.BlockSpec((1, D), lambda i: (0, 0)),
          pl.BlockSpec((1, D), lambda i: (0, 0)),
      ],
      out_specs=pl.BlockSpec((BM, D), lambda i: (i, 0)),
      out_shape=jax.ShapeDtypeStruct((N, D), jnp.float32),
  )(u_part, x, W1, b1.reshape(1, H), W2, b2.reshape(1, D),
    gamma.reshape(1, D), beta.reshape(1, D))


def kernel(x, edge_index, W1, b1, W2, b2, gamma, beta):
  ei = edge_index.astype(jnp.int32)
  pad = EPT_P - EPT
  src_r = jnp.pad(ei[0].reshape(NW, EPT),
                  ((0, 0), (0, pad))).reshape(NW, NCH, CH)
  dst_r = jnp.pad(ei[1].reshape(NW, EPT), ((0, 0), (0, pad)),
                  constant_values=TRASH).reshape(NW, NCH, CH)
  x_q = jnp.rint(x * SCALE).astype(jnp.int16)
  zrows = jnp.zeros((ZR, D), jnp.int16)
  u_part = _sc_segment_sum(x_q, src_r, dst_r, zrows)
  return _mlp_ln(u_part, x, W1, b1, W2, b2, gamma, beta)


# speedup vs baseline: 1.9450x; 1.9450x over previous
import functools

import jax
import jax.numpy as jnp
from jax import lax
from jax.experimental import pallas as pl
from jax.experimental.pallas import tpu as pltpu
from jax.experimental.pallas import tpu_sc as plsc

N = 10000
E = 320000
D = 128
H = 512

NC = 2
NS = 16
DH = D // NC

SCALE = 256.0
CH = 128
EPT = E // NS
NCH = 160
EPT_P = NCH * CH
TRASH = N
ACC_ROWS = 10240
ZR = ACC_ROWS // NS
XR = N // NS


def _sc_segment_sum(x_cat, src_r, dst_r, zrows):
  mesh = plsc.VectorSubcoreMesh(
      core_axis_name="c", subcore_axis_name="s", num_cores=NC,
      num_subcores=NS)

  @functools.partial(
      pl.kernel,
      out_type=jax.ShapeDtypeStruct((NC, ACC_ROWS, DH), jnp.int16),
      mesh=mesh,
      scratch_types=[
          pltpu.VMEM((NCH, CH), jnp.int32),
          pltpu.VMEM((NCH, CH), jnp.int32),
          pltpu.VMEM((CH, DH), jnp.int16),
          pltpu.VMEM((CH, DH), jnp.int16),
          pltpu.VMEM((CH, DH), jnp.int16),
          pltpu.VMEM((CH, DH), jnp.int16),
          pltpu.SemaphoreType.DMA,
          pltpu.SemaphoreType.DMA,
          pltpu.SemaphoreType.DMA,
          pltpu.SemaphoreType.DMA,
          pltpu.SemaphoreType.DMA,
          pltpu.SemaphoreType.DMA,
          pltpu.SemaphoreType.DMA,
          pltpu.SemaphoreType.DMA,
          pltpu.VMEM_SHARED((N, DH), jnp.int16),
          pltpu.VMEM_SHARED((ACC_ROWS, DH), jnp.int16),
      ],
      compiler_params=pltpu.CompilerParams(use_tc_tiling_on_sc=False),
  )
  def seg_sum(x_hbm, src_hbm, dst_hbm, zero_hbm, out_hbm,
              src_v, dst_v, g0, g1, g2, g3,
              gs0, gs1, gs2, gs3, ss0, ss1, ss2, ss3, xs, acc):
    c = lax.axis_index("c")
    s = lax.axis_index("s")
    gb = [g0, g1, g2, g3]
    gsem = [gs0, gs1, gs2, gs3]
    ssem = [ss0, ss1, ss2, ss3]

    pltpu.sync_copy(src_hbm.at[s], src_v)
    pltpu.sync_copy(dst_hbm.at[s], dst_v)
    pltpu.sync_copy(x_hbm.at[c, pl.ds(s * XR, XR)], xs.at[pl.ds(s * XR, XR)])
    pltpu.sync_copy(zero_hbm, acc.at[pl.ds(s * ZR, ZR)])
    plsc.subcore_barrier()

    for t in range(3):
      pltpu.async_copy(xs.at[src_v.at[t]], gb[t], gsem[t])

    def quad(p, carry):
      for t in range(4):
        j = 4 * p + t

        @pl.when(j >= 4)
        def _drain_scatter():
          pltpu.make_async_copy(gb[t], acc.at[dst_v.at[j]], ssem[t]).wait()

        pltpu.make_async_copy(xs.at[src_v.at[j]], gb[t], gsem[t]).wait()
        pltpu.async_copy(gb[t], acc.at[dst_v.at[j]], ssem[t], add=True)

        @pl.when(j + 3 < NCH)
        def _fire_next():
          tn = (t + 3) % 4
          pltpu.async_copy(xs.at[src_v.at[j + 3]], gb[tn], gsem[tn])

      return carry

    lax.fori_loop(0, NCH // 4, quad, None)
    for t in range(4):
      pltpu.make_async_copy(gb[t], acc.at[dst_v.at[NCH - 4 + t]],
                            ssem[t]).wait()
    plsc.subcore_barrier()

    pltpu.sync_copy(acc.at[pl.ds(s * ZR, ZR)],
                    out_hbm.at[c, pl.ds(s * ZR, ZR)])

  return seg_sum(x_cat, src_r, dst_r, zrows)


BM = 1000


def _mlp_body(u_ref, x_ref, w1_ref, b1_ref, w2_ref, b2_ref, g_ref, bt_ref,
              o_ref):
  u = jnp.concatenate([u_ref[0], u_ref[1]], axis=1).astype(jnp.float32)
  u = u * (1.0 / SCALE)
  h1 = jnp.dot(u, w1_ref[...], preferred_element_type=jnp.float32)
  h1 = jnp.maximum(h1 + b1_ref[...], 0.0)
  h = jnp.dot(h1, w2_ref[...], preferred_element_type=jnp.float32)
  y = h + b2_ref[...] + x_ref[...]
  mean = jnp.mean(y, axis=1, keepdims=True)
  yc = y - mean
  var = jnp.mean(yc * yc, axis=1, keepdims=True)
  o_ref[...] = yc * lax.rsqrt(var + 1e-5) * g_ref[...] + bt_ref[...]


def _mlp_ln(u_part, x, W1, b1, W2, b2, gamma, beta):
  grid = (N // BM,)
  return pl.pallas_call(
      _mlp_body,
      grid=grid,
      in_specs=[
          pl.BlockSpec((NC, BM, DH), lambda i: (0, i, 0)),
          pl.BlockSpec((BM, D), lambda i: (i, 0)),
          pl.BlockSpec((D, H), lambda i: (0, 0)),
          pl.BlockSpec((1, H), lambda i: (0, 0)),
          pl.BlockSpec((H, D), lambda i: (0, 0)),
          pl.BlockSpec((1, D), lambda i: (0, 0)),
          pl.BlockSpec((1, D), lambda i: (0, 0)),
          pl.BlockSpec((1, D), lambda i: (0, 0)),
      ],
      out_specs=pl.BlockSpec((BM, D), lambda i: (i, 0)),
      out_shape=jax.ShapeDtypeStruct((N, D), jnp.float32),
  )(u_part, x, W1, b1.reshape(1, H), W2, b2.reshape(1, D),
    gamma.reshape(1, D), beta.reshape(1, D))


def kernel(x, edge_index, W1, b1, W2, b2, gamma, beta):
  ei = edge_index.astype(jnp.int32)
  pad = EPT_P - EPT
  src_r = jnp.pad(ei[0].reshape(NS, EPT),
                  ((0, 0), (0, pad))).reshape(NS, NCH, CH)
  dst_r = jnp.pad(ei[1].reshape(NS, EPT), ((0, 0), (0, pad)),
                  constant_values=TRASH).reshape(NS, NCH, CH)
  x_q = jnp.rint(x * SCALE).astype(jnp.int16)
  x_cat = jnp.stack([x_q[:, :DH], x_q[:, DH:]])
  zrows = jnp.zeros((ZR, DH), jnp.int16)
  u_part = _sc_segment_sum(x_cat, src_r, dst_r, zrows)
  return _mlp_ln(u_part, x, W1, b1, W2, b2, gamma, beta)
